# Initial kernel scaffold; baseline (speedup 1.0000x reference)
import jax
import jax.numpy as jnp
from jax.experimental import pallas as pl


def kernel(chn_llr, beta_logit, edges_var, edges_chk, perms, inv_perms):
    raise NotImplementedError("write your pallas kernel here")


# speedup vs baseline: 3.1621x; 3.1621x over previous
import jax
import jax.numpy as jnp
from jax.experimental import pallas as pl
from jax.experimental.pallas import tpu as pltpu

_N = 255
_NP = 256
_M = 128
_DC = 8
_E = _M * _DC
_T_RRD = 30
_T_BP = 3
_BT = 256


def _rrd_body(beta_ref, e_ref, x_ref, out_ref):
    beta = beta_ref[0, 0]
    x = x_ref[...]
    bt = x.shape[1]
    col = jax.lax.broadcasted_iota(jnp.int32, (_E, _NP), 1)

    def rrd_iter(t, carry):
        cur, ext = carry
        nc = cur + beta * ext
        e = e_ref[t]
        g = (e == col).astype(jnp.float32)
        msg = jnp.zeros((_E, bt), jnp.float32)
        s = jnp.zeros((_NP, bt), jnp.float32)
        for _ in range(_T_BP):
            comb = nc + s
            ge = jax.lax.dot_general(
                g, comb, (((1,), (0,)), ((), ())),
                precision=jax.lax.Precision.HIGHEST,
                preferred_element_type=jnp.float32)
            v2c = ge - msg
            th = jnp.tanh(jnp.clip(0.5 * v2c, -15.0, 15.0))
            la = jnp.log(jnp.abs(th) + 1e-12)
            neg = jnp.where(th < 0, 1.0, 0.0)
            la3 = la.reshape(_M, _DC, bt)
            ng3 = neg.reshape(_M, _DC, bt)
            sum_log = jnp.sum(la3, axis=1, keepdims=True)
            sum_neg = jnp.sum(ng3, axis=1, keepdims=True)
            pm = jnp.exp(jnp.broadcast_to(sum_log, la3.shape).reshape(_E, bt) - la)
            d = jnp.broadcast_to(sum_neg, ng3.shape).reshape(_E, bt) - neg
            par = d - 2.0 * jnp.floor(d * 0.5)
            sg = 1.0 - 2.0 * par
            prod = jnp.clip(sg * pm, -0.999999, 0.999999)
            msg = jnp.log((1.0 + prod) / (1.0 - prod))
            s = jax.lax.dot_general(
                g, msg, (((0,), (0,)), ((), ())),
                precision=jax.lax.Precision.HIGHEST,
                preferred_element_type=jnp.float32)
        out_ref[t] = nc + s
        return nc, s

    jax.lax.fori_loop(0, _T_RRD, rrd_iter, (x, jnp.zeros_like(x)))


def kernel(chn_llr, beta_logit, edges_var, edges_chk, perms, inv_perms):
    del edges_chk, inv_perms
    batch = chn_llr.shape[1]
    beta = jax.nn.sigmoid(beta_logit).reshape(1, 1)
    e_all = perms[:, edges_var][..., None].astype(jnp.int32)
    xp = jnp.concatenate(
        [chn_llr, jnp.zeros((_NP - _N, batch), chn_llr.dtype)], axis=0)
    out = pl.pallas_call(
        _rrd_body,
        grid=(batch // _BT,),
        in_specs=[
            pl.BlockSpec(memory_space=pltpu.SMEM),
            pl.BlockSpec((_T_RRD, _E, 1), lambda i: (0, 0, 0)),
            pl.BlockSpec((_NP, _BT), lambda i: (0, i)),
        ],
        out_specs=pl.BlockSpec((_T_RRD, _NP, _BT), lambda i: (0, 0, i)),
        out_shape=jax.ShapeDtypeStruct((_T_RRD, _NP, batch), jnp.float32),
    )(beta, e_all, xp)
    return out[:, :_N, :]


# speedup vs baseline: 4.5394x; 1.4356x over previous
import jax
import jax.numpy as jnp
from jax.experimental import pallas as pl
from jax.experimental.pallas import tpu as pltpu

_N = 255
_NP = 256
_M = 128
_DC = 8
_E = _M * _DC
_T_RRD = 30
_T_BP = 3
_BT = 512


def _rrd_body(beta_ref, e_ref, x_ref, out_ref):
    beta = beta_ref[0, 0]
    x = x_ref[...]
    bt = x.shape[1]
    col = jax.lax.broadcasted_iota(jnp.int32, (_E, _NP), 1)

    def rrd_iter(t, carry):
        cur, ext = carry
        nc = cur + beta * ext
        e = e_ref[t]
        g = (e == col).astype(jnp.float32)
        msg = jnp.zeros((_E, bt), jnp.float32)
        s = jnp.zeros((_NP, bt), jnp.float32)
        for _ in range(_T_BP):
            comb = nc + s
            ge = jax.lax.dot_general(
                g, comb, (((1,), (0,)), ((), ())),
                precision=jax.lax.Precision.HIGHEST,
                preferred_element_type=jnp.float32)
            v2c = ge - msg
            th = jnp.tanh(jnp.clip(0.5 * v2c, -15.0, 15.0))
            la = jnp.log(jnp.abs(th) + 1e-12)
            neg = jnp.where(th < 0, 1.0, 0.0)
            la3 = la.reshape(_M, _DC, bt)
            ng3 = neg.reshape(_M, _DC, bt)
            sum_log = jnp.sum(la3, axis=1, keepdims=True)
            sum_neg = jnp.sum(ng3, axis=1, keepdims=True)
            pm = jnp.exp(jnp.broadcast_to(sum_log, la3.shape).reshape(_E, bt) - la)
            d = jnp.broadcast_to(sum_neg, ng3.shape).reshape(_E, bt) - neg
            par = d - 2.0 * jnp.floor(d * 0.5)
            sg = 1.0 - 2.0 * par
            prod = jnp.clip(sg * pm, -0.999999, 0.999999)
            msg = jnp.log((1.0 + prod) / (1.0 - prod))
            s = jax.lax.dot_general(
                g, msg, (((0,), (0,)), ((), ())),
                precision=jax.lax.Precision.HIGHEST,
                preferred_element_type=jnp.float32)
        out_ref[t] = nc + s
        return nc, s

    jax.lax.fori_loop(0, _T_RRD, rrd_iter, (x, jnp.zeros_like(x)))


def kernel(chn_llr, beta_logit, edges_var, edges_chk, perms, inv_perms):
    del edges_chk, inv_perms
    batch = chn_llr.shape[1]
    beta = jax.nn.sigmoid(beta_logit).reshape(1, 1)
    e_all = perms[:, edges_var][..., None].astype(jnp.int32)
    xp = jnp.concatenate(
        [chn_llr, jnp.zeros((_NP - _N, batch), chn_llr.dtype)], axis=0)
    out = pl.pallas_call(
        _rrd_body,
        grid=(batch // _BT,),
        in_specs=[
            pl.BlockSpec(memory_space=pltpu.SMEM),
            pl.BlockSpec((_T_RRD, _E, 1), lambda i: (0, 0, 0)),
            pl.BlockSpec((_NP, _BT), lambda i: (0, i)),
        ],
        out_specs=pl.BlockSpec((_T_RRD, _NP, _BT), lambda i: (0, 0, i)),
        out_shape=jax.ShapeDtypeStruct((_T_RRD, _NP, batch), jnp.float32),
    )(beta, e_all, xp)
    return out[:, :_N, :]


# speedup vs baseline: 4.5418x; 1.0005x over previous
import jax
import jax.numpy as jnp
from jax.experimental import pallas as pl
from jax.experimental.pallas import tpu as pltpu

_N = 255
_NP = 256
_M = 128
_DC = 8
_E = _M * _DC
_T_RRD = 30
_T_BP = 3
_BT = 512


def _rrd_body(beta_ref, e_ref, x_ref, out_ref):
    beta = beta_ref[0, 0]
    x = x_ref[...]
    bt = x.shape[1]
    row = jax.lax.broadcasted_iota(jnp.int32, (_NP, _E), 0)

    def rrd_iter(t, carry):
        cur, ext = carry
        nc = cur + beta * ext
        e = e_ref[t]
        gt = (e == row).astype(jnp.float32)
        msg = jnp.zeros((_E, bt), jnp.float32)
        s = jnp.zeros((_NP, bt), jnp.float32)
        for _ in range(_T_BP):
            comb = nc + s
            ge = jax.lax.dot_general(
                gt, comb, (((0,), (0,)), ((), ())),
                precision=jax.lax.Precision.HIGHEST,
                preferred_element_type=jnp.float32)
            v2c = ge - msg
            th = jnp.tanh(jnp.clip(0.5 * v2c, -15.0, 15.0))
            la = jnp.log(jnp.abs(th) + 1e-12)
            neg = jnp.where(th < 0, 1.0, 0.0)
            la3 = la.reshape(_M, _DC, bt)
            ng3 = neg.reshape(_M, _DC, bt)
            sum_log = jnp.sum(la3, axis=1, keepdims=True)
            sum_neg = jnp.sum(ng3, axis=1, keepdims=True)
            pm = jnp.exp(jnp.broadcast_to(sum_log, la3.shape).reshape(_E, bt) - la)
            d = jnp.broadcast_to(sum_neg, ng3.shape).reshape(_E, bt) - neg
            par = d - 2.0 * jnp.floor(d * 0.5)
            sg = 1.0 - 2.0 * par
            prod = jnp.clip(sg * pm, -0.999999, 0.999999)
            msg = jnp.log((1.0 + prod) / (1.0 - prod))
            s = jax.lax.dot_general(
                gt, msg, (((1,), (0,)), ((), ())),
                precision=jax.lax.Precision.HIGHEST,
                preferred_element_type=jnp.float32)
        out_ref[t] = nc + s
        return nc, s

    jax.lax.fori_loop(0, _T_RRD, rrd_iter, (x, jnp.zeros_like(x)))


def kernel(chn_llr, beta_logit, edges_var, edges_chk, perms, inv_perms):
    del edges_chk, inv_perms
    batch = chn_llr.shape[1]
    beta = jax.nn.sigmoid(beta_logit).reshape(1, 1)
    e_all = perms[:, edges_var][:, None, :].astype(jnp.int32)
    xp = jnp.concatenate(
        [chn_llr, jnp.zeros((_NP - _N, batch), chn_llr.dtype)], axis=0)
    out = pl.pallas_call(
        _rrd_body,
        grid=(batch // _BT,),
        in_specs=[
            pl.BlockSpec(memory_space=pltpu.SMEM),
            pl.BlockSpec((_T_RRD, 1, _E), lambda i: (0, 0, 0)),
            pl.BlockSpec((_NP, _BT), lambda i: (0, i)),
        ],
        out_specs=pl.BlockSpec((_T_RRD, _NP, _BT), lambda i: (0, 0, i)),
        out_shape=jax.ShapeDtypeStruct((_T_RRD, _NP, batch), jnp.float32),
    )(beta, e_all, xp)
    return out[:, :_N, :]


# speedup vs baseline: 5.6442x; 1.2427x over previous
import jax
import jax.numpy as jnp
from jax.experimental import pallas as pl
from jax.experimental.pallas import tpu as pltpu

_N = 255
_NP = 256
_M = 128
_DC = 8
_E = _M * _DC
_T_RRD = 30
_T_BP = 3
_BT = 512


def _split3(x):
    hi = x.astype(jnp.bfloat16)
    r1 = x - hi.astype(jnp.float32)
    mid = r1.astype(jnp.bfloat16)
    lo = (r1 - mid.astype(jnp.float32)).astype(jnp.bfloat16)
    return hi, mid, lo


def _dot3(g, x, dims):
    acc = None
    for part in _split3(x):
        p = jax.lax.dot_general(g, part, (dims, ((), ())),
                                preferred_element_type=jnp.float32)
        acc = p if acc is None else acc + p
    return acc


def _rrd_body(beta_ref, e_ref, x_ref, out_ref):
    beta = beta_ref[0, 0]
    x = x_ref[...]
    bt = x.shape[1]
    row = jax.lax.broadcasted_iota(jnp.int32, (_NP, _E), 0)

    def rrd_iter(t, carry):
        cur, ext = carry
        nc = cur + beta * ext
        e = e_ref[t]
        gt = (e == row).astype(jnp.bfloat16)
        msg = jnp.zeros((_E, bt), jnp.float32)
        s = jnp.zeros((_NP, bt), jnp.float32)
        for _ in range(_T_BP):
            comb = nc + s
            ge = _dot3(gt, comb, ((0,), (0,)))
            v2c = ge - msg
            th = jnp.tanh(jnp.clip(0.5 * v2c, -15.0, 15.0))
            la = jnp.log(jnp.abs(th) + 1e-12)
            neg = jnp.where(th < 0, 1.0, 0.0)
            la3 = la.reshape(_M, _DC, bt)
            ng3 = neg.reshape(_M, _DC, bt)
            sum_log = jnp.sum(la3, axis=1, keepdims=True)
            sum_neg = jnp.sum(ng3, axis=1, keepdims=True)
            pm = jnp.exp(jnp.broadcast_to(sum_log, la3.shape).reshape(_E, bt) - la)
            d = jnp.broadcast_to(sum_neg, ng3.shape).reshape(_E, bt) - neg
            par = d - 2.0 * jnp.floor(d * 0.5)
            sg = 1.0 - 2.0 * par
            prod = jnp.clip(sg * pm, -0.999999, 0.999999)
            msg = jnp.log((1.0 + prod) / (1.0 - prod))
            s = _dot3(gt, msg, ((1,), (0,)))
        out_ref[t] = nc + s
        return nc, s

    jax.lax.fori_loop(0, _T_RRD, rrd_iter, (x, jnp.zeros_like(x)))


def kernel(chn_llr, beta_logit, edges_var, edges_chk, perms, inv_perms):
    del edges_chk, inv_perms
    batch = chn_llr.shape[1]
    beta = jax.nn.sigmoid(beta_logit).reshape(1, 1)
    e_all = perms[:, edges_var][:, None, :].astype(jnp.int32)
    xp = jnp.concatenate(
        [chn_llr, jnp.zeros((_NP - _N, batch), chn_llr.dtype)], axis=0)
    out = pl.pallas_call(
        _rrd_body,
        grid=(batch // _BT,),
        in_specs=[
            pl.BlockSpec(memory_space=pltpu.SMEM),
            pl.BlockSpec((_T_RRD, 1, _E), lambda i: (0, 0, 0)),
            pl.BlockSpec((_NP, _BT), lambda i: (0, i)),
        ],
        out_specs=pl.BlockSpec((_T_RRD, _NP, _BT), lambda i: (0, 0, i)),
        out_shape=jax.ShapeDtypeStruct((_T_RRD, _NP, batch), jnp.float32),
    )(beta, e_all, xp)
    return out[:, :_N, :]
